# Initial kernel scaffold; baseline (speedup 1.0000x reference)
import jax
import jax.numpy as jnp
from jax.experimental import pallas as pl


def kernel(x, edge_index, W1, b1, W2, b2):
    raise NotImplementedError("write your pallas kernel here")


# speedup vs baseline: 22.4964x; 22.4964x over previous
import functools

import jax
import jax.numpy as jnp
from jax import lax
from jax.experimental import pallas as pl
from jax.experimental.pallas import tpu as pltpu
from jax.experimental.pallas import tpu_sc as plsc

N = 10000
F0 = 128
F1 = 16
F2 = 2
E = 320000

NPAD = 10240
DUMMY = 10000
EW = 128
ROWS = 2560
EPAD = ROWS * EW

_NC = 2
_NS = 16
_RPT = ROWS // (_NC * _NS)
_SL = NPAD // _NS

_BR = 1024


def _make_segsum():
    mesh = plsc.VectorSubcoreMesh(core_axis_name="c", subcore_axis_name="s")

    @functools.partial(
        pl.kernel,
        out_type=jax.ShapeDtypeStruct((_NC, NPAD, F1), jnp.float32),
        mesh=mesh,
        compiler_params=pltpu.CompilerParams(use_tc_tiling_on_sc=False),
        scratch_types=[
            pltpu.VMEM((_RPT, EW), jnp.int32),
            pltpu.VMEM((_RPT, EW), jnp.int32),
            pltpu.VMEM((EW, F1), jnp.float32),
            pltpu.VMEM_SHARED((NPAD, F1), jnp.float32),
        ],
    )
    def segsum(vals_hbm, src_hbm, dst_hbm, zeros_hbm, out_hbm,
               src_v, dst_v, rows_v, acc_sh):
        c = lax.axis_index("c")
        s = lax.axis_index("s")
        wid = s * _NC + c
        pltpu.sync_copy(src_hbm.at[pl.ds(wid * _RPT, _RPT)], src_v)
        pltpu.sync_copy(dst_hbm.at[pl.ds(wid * _RPT, _RPT)], dst_v)
        pltpu.sync_copy(zeros_hbm.at[pl.ds(s * _SL, _SL)],
                        acc_sh.at[pl.ds(s * _SL, _SL)])
        plsc.subcore_barrier()

        @pl.loop(0, _RPT)
        def _(j):
            pltpu.sync_copy(vals_hbm.at[src_v.at[j]], rows_v)
            pltpu.sync_copy(rows_v, acc_sh.at[dst_v.at[j]], add=True)

        plsc.subcore_barrier()
        pltpu.sync_copy(acc_sh.at[pl.ds(s * _SL, _SL)],
                        out_hbm.at[c, pl.ds(s * _SL, _SL)])

    return segsum


_segsum = _make_segsum()


def _mm1_body(x_ref, w_ref, o_ref):
    o_ref[...] = jnp.dot(x_ref[...], w_ref[...],
                         preferred_element_type=jnp.float32)


def _scale_body(p_ref, h1_ref, dis_ref, h1s_ref):
    deg = p_ref[0] + p_ref[1] + 1.0
    dis = lax.rsqrt(deg)
    dis_ref[...] = dis
    h1s_ref[...] = h1_ref[...] * dis


def _layer1_body(p_ref, h1s_ref, dis_ref, b1_ref, r_ref):
    dis = dis_ref[...]
    o1 = dis * (p_ref[0] + p_ref[1] + h1s_ref[...]) + b1_ref[...]
    r_ref[...] = dis * jnp.maximum(o1, 0.0)


def _layer2_body(p_ref, r_ref, dis_ref, w2_ref, b2_ref, o_ref):
    t = dis_ref[...] * (p_ref[0] + p_ref[1] + r_ref[...])
    o2 = jnp.dot(t, w2_ref[...], preferred_element_type=jnp.float32) + b2_ref[...]
    m = jnp.max(o2, axis=1, keepdims=True)
    lse = m + jnp.log(jnp.sum(jnp.exp(o2 - m), axis=1, keepdims=True))
    o_ref[...] = o2 - lse


def _row_spec(width):
    return pl.BlockSpec((_BR, width), lambda i: (i, 0))


def _pair_spec(width):
    return pl.BlockSpec((_NC, _BR, width), lambda i: (0, i, 0))


def _const_spec(shape):
    return pl.BlockSpec(shape, lambda i: tuple(0 for _ in shape))


_GRID = (NPAD // _BR,)


def kernel(x, edge_index, W1, b1, W2, b2):
    ei = edge_index.astype(jnp.int32)
    src = jnp.pad(ei[0], (0, EPAD - E), constant_values=DUMMY).reshape(ROWS, EW)
    dst = jnp.pad(ei[1], (0, EPAD - E), constant_values=DUMMY).reshape(ROWS, EW)
    x_pad = jnp.pad(x, ((0, NPAD - N), (0, 0)))
    zeros = jnp.zeros((NPAD, F1), jnp.float32)
    ones = jnp.ones((NPAD, F1), jnp.float32)
    b1r = b1.reshape(1, F1)
    b2r = b2.reshape(1, F2)

    degp = _segsum(ones, dst, dst, zeros)

    h1 = pl.pallas_call(
        _mm1_body,
        grid=_GRID,
        in_specs=[_row_spec(F0), _const_spec((F0, F1))],
        out_specs=_row_spec(F1),
        out_shape=jax.ShapeDtypeStruct((NPAD, F1), jnp.float32),
    )(x_pad, W1)

    dis, h1s = pl.pallas_call(
        _scale_body,
        grid=_GRID,
        in_specs=[_pair_spec(F1), _row_spec(F1)],
        out_specs=[_row_spec(F1), _row_spec(F1)],
        out_shape=[jax.ShapeDtypeStruct((NPAD, F1), jnp.float32),
                   jax.ShapeDtypeStruct((NPAD, F1), jnp.float32)],
    )(degp, h1)

    a1p = _segsum(h1s, src, dst, zeros)

    r = pl.pallas_call(
        _layer1_body,
        grid=_GRID,
        in_specs=[_pair_spec(F1), _row_spec(F1), _row_spec(F1),
                  _const_spec((1, F1))],
        out_specs=_row_spec(F1),
        out_shape=jax.ShapeDtypeStruct((NPAD, F1), jnp.float32),
    )(a1p, h1s, dis, b1r)

    a2p = _segsum(r, src, dst, zeros)

    out = pl.pallas_call(
        _layer2_body,
        grid=_GRID,
        in_specs=[_pair_spec(F1), _row_spec(F1), _row_spec(F1),
                  _const_spec((F1, F2)), _const_spec((1, F2))],
        out_specs=_row_spec(F2),
        out_shape=jax.ShapeDtypeStruct((NPAD, F2), jnp.float32),
    )(a2p, r, dis, W2, b2r)

    return out[:N]


# speedup vs baseline: 36.0110x; 1.6007x over previous
import functools

import jax
import jax.numpy as jnp
from jax import lax
from jax.experimental import pallas as pl
from jax.experimental.pallas import tpu as pltpu
from jax.experimental.pallas import tpu_sc as plsc

N = 10000
F0 = 128
F1 = 16
F2 = 2
E = 320000

NPAD = 10240
DUMMY = 10000
EW = 128
ROWS = 2560
EPAD = ROWS * EW

_NC = 2
_NS = 16
_RPT = ROWS // (_NC * _NS)
_SL = NPAD // _NS

_BR = 1024
_NB = 8


_MESH = plsc.VectorSubcoreMesh(core_axis_name="c", subcore_axis_name="s")
_SC_PARAMS = pltpu.CompilerParams(use_tc_tiling_on_sc=False)


def _make_segsum():
    @functools.partial(
        pl.kernel,
        out_type=jax.ShapeDtypeStruct((_NC, NPAD, F1), jnp.float32),
        mesh=_MESH,
        compiler_params=_SC_PARAMS,
        scratch_types=[
            pltpu.VMEM((_RPT, EW), jnp.int32),
            pltpu.VMEM((_RPT, EW), jnp.int32),
            pltpu.VMEM((_NB, EW, F1), jnp.float32),
            pltpu.VMEM_SHARED((NPAD, F1), jnp.float32),
            pltpu.SemaphoreType.DMA,
            pltpu.SemaphoreType.DMA,
        ],
    )
    def segsum(vals_hbm, src_hbm, dst_hbm, zeros_hbm, out_hbm,
               src_v, dst_v, rows_v, acc_sh, gsem, ssem):
        c = lax.axis_index("c")
        s = lax.axis_index("s")
        wid = s * _NC + c
        pltpu.sync_copy(src_hbm.at[pl.ds(wid * _RPT, _RPT)], src_v)
        pltpu.sync_copy(dst_hbm.at[pl.ds(wid * _RPT, _RPT)], dst_v)
        pltpu.sync_copy(zeros_hbm.at[pl.ds(s * _SL, _SL)],
                        acc_sh.at[pl.ds(s * _SL, _SL)])
        plsc.subcore_barrier()

        @pl.loop(0, _RPT, step=_NB)
        def _(j0):
            gs = [pltpu.async_copy(vals_hbm.at[src_v.at[j0 + b]],
                                   rows_v.at[b], gsem)
                  for b in range(_NB)]
            ss = []
            for b in range(_NB):
                gs[b].wait()
                ss.append(pltpu.async_copy(rows_v.at[b],
                                           acc_sh.at[dst_v.at[j0 + b]],
                                           ssem, add=True))
            for h in ss:
                h.wait()

        plsc.subcore_barrier()
        pltpu.sync_copy(acc_sh.at[pl.ds(s * _SL, _SL)],
                        out_hbm.at[c, pl.ds(s * _SL, _SL)])

    return segsum


def _make_degree():
    @functools.partial(
        pl.kernel,
        out_type=jax.ShapeDtypeStruct((_NC, NPAD, F1), jnp.float32),
        mesh=_MESH,
        compiler_params=_SC_PARAMS,
        scratch_types=[
            pltpu.VMEM((_RPT, EW), jnp.int32),
            pltpu.VMEM((EW, F1), jnp.float32),
            pltpu.VMEM_SHARED((NPAD, F1), jnp.float32),
            pltpu.SemaphoreType.DMA,
        ],
    )
    def degree(ones_hbm, dst_hbm, zeros_hbm, out_hbm,
               dst_v, ones_v, acc_sh, ssem):
        c = lax.axis_index("c")
        s = lax.axis_index("s")
        wid = s * _NC + c
        pltpu.sync_copy(dst_hbm.at[pl.ds(wid * _RPT, _RPT)], dst_v)
        pltpu.sync_copy(ones_hbm.at[pl.ds(0, EW)], ones_v)
        pltpu.sync_copy(zeros_hbm.at[pl.ds(s * _SL, _SL)],
                        acc_sh.at[pl.ds(s * _SL, _SL)])
        plsc.subcore_barrier()

        @pl.loop(0, _RPT, step=_NB)
        def _(j0):
            ss = [pltpu.async_copy(ones_v, acc_sh.at[dst_v.at[j0 + b]],
                                   ssem, add=True)
                  for b in range(_NB)]
            for h in ss:
                h.wait()

        plsc.subcore_barrier()
        pltpu.sync_copy(acc_sh.at[pl.ds(s * _SL, _SL)],
                        out_hbm.at[c, pl.ds(s * _SL, _SL)])

    return degree


_segsum = _make_segsum()
_degree = _make_degree()


def _mm1_body(x_ref, w_ref, o_ref):
    o_ref[...] = jnp.dot(x_ref[...], w_ref[...],
                         preferred_element_type=jnp.float32)


def _scale_body(p_ref, h1_ref, dis_ref, h1s_ref):
    deg = p_ref[0] + p_ref[1] + 1.0
    dis = lax.rsqrt(deg)
    dis_ref[...] = dis
    h1s_ref[...] = h1_ref[...] * dis


def _layer1_body(p_ref, h1s_ref, dis_ref, b1_ref, r_ref):
    dis = dis_ref[...]
    o1 = dis * (p_ref[0] + p_ref[1] + h1s_ref[...]) + b1_ref[...]
    r_ref[...] = dis * jnp.maximum(o1, 0.0)


def _layer2_body(p_ref, r_ref, dis_ref, w2_ref, b2_ref, o_ref):
    t = dis_ref[...] * (p_ref[0] + p_ref[1] + r_ref[...])
    o2 = jnp.dot(t, w2_ref[...], preferred_element_type=jnp.float32) + b2_ref[...]
    m = jnp.max(o2, axis=1, keepdims=True)
    lse = m + jnp.log(jnp.sum(jnp.exp(o2 - m), axis=1, keepdims=True))
    o_ref[...] = o2 - lse


def _row_spec(width):
    return pl.BlockSpec((_BR, width), lambda i: (i, 0))


def _pair_spec(width):
    return pl.BlockSpec((_NC, _BR, width), lambda i: (0, i, 0))


def _const_spec(shape):
    return pl.BlockSpec(shape, lambda i: tuple(0 for _ in shape))


_GRID = (NPAD // _BR,)


def kernel(x, edge_index, W1, b1, W2, b2):
    ei = edge_index.astype(jnp.int32)
    src = jnp.pad(ei[0], (0, EPAD - E), constant_values=DUMMY).reshape(ROWS, EW)
    dst = jnp.pad(ei[1], (0, EPAD - E), constant_values=DUMMY).reshape(ROWS, EW)
    x_pad = jnp.pad(x, ((0, NPAD - N), (0, 0)))
    zeros = jnp.zeros((NPAD, F1), jnp.float32)
    ones = jnp.ones((NPAD, F1), jnp.float32)
    b1r = b1.reshape(1, F1)
    b2r = b2.reshape(1, F2)

    degp = _degree(ones, dst, zeros)

    h1 = pl.pallas_call(
        _mm1_body,
        grid=_GRID,
        in_specs=[_row_spec(F0), _const_spec((F0, F1))],
        out_specs=_row_spec(F1),
        out_shape=jax.ShapeDtypeStruct((NPAD, F1), jnp.float32),
    )(x_pad, W1)

    dis, h1s = pl.pallas_call(
        _scale_body,
        grid=_GRID,
        in_specs=[_pair_spec(F1), _row_spec(F1)],
        out_specs=[_row_spec(F1), _row_spec(F1)],
        out_shape=[jax.ShapeDtypeStruct((NPAD, F1), jnp.float32),
                   jax.ShapeDtypeStruct((NPAD, F1), jnp.float32)],
    )(degp, h1)

    a1p = _segsum(h1s, src, dst, zeros)

    r = pl.pallas_call(
        _layer1_body,
        grid=_GRID,
        in_specs=[_pair_spec(F1), _row_spec(F1), _row_spec(F1),
                  _const_spec((1, F1))],
        out_specs=_row_spec(F1),
        out_shape=jax.ShapeDtypeStruct((NPAD, F1), jnp.float32),
    )(a1p, h1s, dis, b1r)

    a2p = _segsum(r, src, dst, zeros)

    out = pl.pallas_call(
        _layer2_body,
        grid=_GRID,
        in_specs=[_pair_spec(F1), _row_spec(F1), _row_spec(F1),
                  _const_spec((F1, F2)), _const_spec((1, F2))],
        out_specs=_row_spec(F2),
        out_shape=jax.ShapeDtypeStruct((NPAD, F2), jnp.float32),
    )(a2p, r, dis, W2, b2r)

    return out[:N]


# speedup vs baseline: 51.8164x; 1.4389x over previous
import functools

import jax
import jax.numpy as jnp
from jax import lax
from jax.experimental import pallas as pl
from jax.experimental.pallas import tpu as pltpu
from jax.experimental.pallas import tpu_sc as plsc

N = 10000
F0 = 128
F1 = 16
F2 = 2
E = 320000

NPAD = 10240
DUMMY = 10000
EW = 128
ROWS = 2560
EPAD = ROWS * EW

_NC = 2
_NS = 16
_RPT = ROWS // (_NC * _NS)
_SL = NPAD // _NS

_BR = 1024
_NB = 8


_MESH = plsc.VectorSubcoreMesh(core_axis_name="c", subcore_axis_name="s")
_SC_PARAMS = pltpu.CompilerParams(use_tc_tiling_on_sc=False)


def _make_segsum():
    @functools.partial(
        pl.kernel,
        out_type=jax.ShapeDtypeStruct((_NC, NPAD, F1), jnp.float32),
        mesh=_MESH,
        compiler_params=_SC_PARAMS,
        scratch_types=[
            pltpu.VMEM((_RPT, EW), jnp.int32),
            pltpu.VMEM((_RPT, EW), jnp.int32),
            pltpu.VMEM((_NB, EW, F1), jnp.float32),
            pltpu.VMEM_SHARED((NPAD, F1), jnp.float32),
            pltpu.VMEM_SHARED((NPAD, F1), jnp.float32),
            pltpu.SemaphoreType.DMA,
            pltpu.SemaphoreType.DMA,
        ],
    )
    def segsum(vals_hbm, src_hbm, dst_hbm, zeros_hbm, out_hbm,
               src_v, dst_v, rows_v, vals_sh, acc_sh, gsem, ssem):
        c = lax.axis_index("c")
        s = lax.axis_index("s")
        wid = s * _NC + c
        pltpu.sync_copy(src_hbm.at[pl.ds(wid * _RPT, _RPT)], src_v)
        pltpu.sync_copy(dst_hbm.at[pl.ds(wid * _RPT, _RPT)], dst_v)
        pltpu.sync_copy(vals_hbm.at[pl.ds(s * _SL, _SL)],
                        vals_sh.at[pl.ds(s * _SL, _SL)])
        pltpu.sync_copy(zeros_hbm.at[pl.ds(s * _SL, _SL)],
                        acc_sh.at[pl.ds(s * _SL, _SL)])
        plsc.subcore_barrier()

        @pl.loop(0, _RPT, step=_NB)
        def _(j0):
            gs = [pltpu.async_copy(vals_sh.at[src_v.at[j0 + b]],
                                   rows_v.at[b], gsem)
                  for b in range(_NB)]
            ss = []
            for b in range(_NB):
                gs[b].wait()
                ss.append(pltpu.async_copy(rows_v.at[b],
                                           acc_sh.at[dst_v.at[j0 + b]],
                                           ssem, add=True))
            for h in ss:
                h.wait()

        plsc.subcore_barrier()
        pltpu.sync_copy(acc_sh.at[pl.ds(s * _SL, _SL)],
                        out_hbm.at[c, pl.ds(s * _SL, _SL)])

    return segsum


def _make_degree():
    @functools.partial(
        pl.kernel,
        out_type=jax.ShapeDtypeStruct((_NC, NPAD, F1), jnp.float32),
        mesh=_MESH,
        compiler_params=_SC_PARAMS,
        scratch_types=[
            pltpu.VMEM((_RPT, EW), jnp.int32),
            pltpu.VMEM((EW, F1), jnp.float32),
            pltpu.VMEM_SHARED((NPAD, F1), jnp.float32),
            pltpu.SemaphoreType.DMA,
        ],
    )
    def degree(ones_hbm, dst_hbm, zeros_hbm, out_hbm,
               dst_v, ones_v, acc_sh, ssem):
        c = lax.axis_index("c")
        s = lax.axis_index("s")
        wid = s * _NC + c
        pltpu.sync_copy(dst_hbm.at[pl.ds(wid * _RPT, _RPT)], dst_v)
        pltpu.sync_copy(ones_hbm.at[pl.ds(0, EW)], ones_v)
        pltpu.sync_copy(zeros_hbm.at[pl.ds(s * _SL, _SL)],
                        acc_sh.at[pl.ds(s * _SL, _SL)])
        plsc.subcore_barrier()

        @pl.loop(0, _RPT, step=_NB)
        def _(j0):
            ss = [pltpu.async_copy(ones_v, acc_sh.at[dst_v.at[j0 + b]],
                                   ssem, add=True)
                  for b in range(_NB)]
            for h in ss:
                h.wait()

        plsc.subcore_barrier()
        pltpu.sync_copy(acc_sh.at[pl.ds(s * _SL, _SL)],
                        out_hbm.at[c, pl.ds(s * _SL, _SL)])

    return degree


_segsum = _make_segsum()
_degree = _make_degree()


def _mm1_body(x_ref, w_ref, o_ref):
    o_ref[...] = jnp.dot(x_ref[...], w_ref[...],
                         preferred_element_type=jnp.float32)


def _scale_body(p_ref, h1_ref, dis_ref, h1s_ref):
    deg = p_ref[0] + p_ref[1] + 1.0
    dis = lax.rsqrt(deg)
    dis_ref[...] = dis
    h1s_ref[...] = h1_ref[...] * dis


def _layer1_body(p_ref, h1s_ref, dis_ref, b1_ref, r_ref):
    dis = dis_ref[...]
    o1 = dis * (p_ref[0] + p_ref[1] + h1s_ref[...]) + b1_ref[...]
    r_ref[...] = dis * jnp.maximum(o1, 0.0)


def _layer2_body(p_ref, r_ref, dis_ref, w2_ref, b2_ref, o_ref):
    t = dis_ref[...] * (p_ref[0] + p_ref[1] + r_ref[...])
    o2 = jnp.dot(t, w2_ref[...], preferred_element_type=jnp.float32) + b2_ref[...]
    m = jnp.max(o2, axis=1, keepdims=True)
    lse = m + jnp.log(jnp.sum(jnp.exp(o2 - m), axis=1, keepdims=True))
    o_ref[...] = o2 - lse


def _row_spec(width):
    return pl.BlockSpec((_BR, width), lambda i: (i, 0))


def _pair_spec(width):
    return pl.BlockSpec((_NC, _BR, width), lambda i: (0, i, 0))


def _const_spec(shape):
    return pl.BlockSpec(shape, lambda i: tuple(0 for _ in shape))


_GRID = (NPAD // _BR,)


def kernel(x, edge_index, W1, b1, W2, b2):
    ei = edge_index.astype(jnp.int32)
    src = jnp.pad(ei[0], (0, EPAD - E), constant_values=DUMMY).reshape(ROWS, EW)
    dst = jnp.pad(ei[1], (0, EPAD - E), constant_values=DUMMY).reshape(ROWS, EW)
    x_pad = jnp.pad(x, ((0, NPAD - N), (0, 0)))
    zeros = jnp.zeros((NPAD, F1), jnp.float32)
    ones = jnp.ones((NPAD, F1), jnp.float32)
    b1r = b1.reshape(1, F1)
    b2r = b2.reshape(1, F2)

    degp = _degree(ones, dst, zeros)

    h1 = pl.pallas_call(
        _mm1_body,
        grid=_GRID,
        in_specs=[_row_spec(F0), _const_spec((F0, F1))],
        out_specs=_row_spec(F1),
        out_shape=jax.ShapeDtypeStruct((NPAD, F1), jnp.float32),
    )(x_pad, W1)

    dis, h1s = pl.pallas_call(
        _scale_body,
        grid=_GRID,
        in_specs=[_pair_spec(F1), _row_spec(F1)],
        out_specs=[_row_spec(F1), _row_spec(F1)],
        out_shape=[jax.ShapeDtypeStruct((NPAD, F1), jnp.float32),
                   jax.ShapeDtypeStruct((NPAD, F1), jnp.float32)],
    )(degp, h1)

    a1p = _segsum(h1s, src, dst, zeros)

    r = pl.pallas_call(
        _layer1_body,
        grid=_GRID,
        in_specs=[_pair_spec(F1), _row_spec(F1), _row_spec(F1),
                  _const_spec((1, F1))],
        out_specs=_row_spec(F1),
        out_shape=jax.ShapeDtypeStruct((NPAD, F1), jnp.float32),
    )(a1p, h1s, dis, b1r)

    a2p = _segsum(r, src, dst, zeros)

    out = pl.pallas_call(
        _layer2_body,
        grid=_GRID,
        in_specs=[_pair_spec(F1), _row_spec(F1), _row_spec(F1),
                  _const_spec((F1, F2)), _const_spec((1, F2))],
        out_specs=_row_spec(F2),
        out_shape=jax.ShapeDtypeStruct((NPAD, F2), jnp.float32),
    )(a2p, r, dis, W2, b2r)

    return out[:N]


# speedup vs baseline: 72.1767x; 1.3929x over previous
import functools

import jax
import jax.numpy as jnp
from jax import lax
from jax.experimental import pallas as pl
from jax.experimental.pallas import tpu as pltpu
from jax.experimental.pallas import tpu_sc as plsc

N = 10000
F0 = 128
F1 = 16
F2 = 2
E = 320000

NPAD = 10240
DUMMY = 10000
EW = 128
ROWS = 2560
EPAD = ROWS * EW

_NC = 2
_NS = 16
_RPT = ROWS // (_NC * _NS)
_SL = NPAD // _NS

_BR = 1024
_NB = 8


_MESH = plsc.VectorSubcoreMesh(core_axis_name="c", subcore_axis_name="s")
_SC_PARAMS = pltpu.CompilerParams(use_tc_tiling_on_sc=False)


def _make_segsum():
    @functools.partial(
        pl.kernel,
        out_type=jax.ShapeDtypeStruct((_NC, NPAD, F1), jnp.float32),
        mesh=_MESH,
        compiler_params=_SC_PARAMS,
        scratch_types=[
            pltpu.VMEM((_RPT, EW), jnp.int32),
            pltpu.VMEM((_RPT, EW), jnp.int32),
            pltpu.VMEM((_NB, EW, F1), jnp.float32),
            pltpu.VMEM_SHARED((NPAD, F1), jnp.float32),
            pltpu.VMEM_SHARED((NPAD, F1), jnp.float32),
            pltpu.SemaphoreType.DMA,
            pltpu.SemaphoreType.DMA,
        ],
    )
    def segsum(vals_hbm, src_hbm, dst_hbm, zeros_hbm, out_hbm,
               src_v, dst_v, rows_v, vals_sh, acc_sh, gsem, ssem):
        c = lax.axis_index("c")
        s = lax.axis_index("s")
        wid = s * _NC + c
        pltpu.sync_copy(src_hbm.at[pl.ds(wid * _RPT, _RPT)], src_v)
        pltpu.sync_copy(dst_hbm.at[pl.ds(wid * _RPT, _RPT)], dst_v)
        pltpu.sync_copy(vals_hbm.at[pl.ds(s * _SL, _SL)],
                        vals_sh.at[pl.ds(s * _SL, _SL)])
        pltpu.sync_copy(zeros_hbm.at[pl.ds(s * _SL, _SL)],
                        acc_sh.at[pl.ds(s * _SL, _SL)])
        plsc.subcore_barrier()

        @pl.loop(0, _RPT, step=_NB)
        def _(j0):
            gs = [pltpu.async_copy(vals_sh.at[src_v.at[j0 + b]],
                                   rows_v.at[b], gsem)
                  for b in range(_NB)]
            ss = []
            for b in range(_NB):
                gs[b].wait()
                ss.append(pltpu.async_copy(rows_v.at[b],
                                           acc_sh.at[dst_v.at[j0 + b]],
                                           ssem, add=True))
            for h in ss:
                h.wait()

        plsc.subcore_barrier()
        pltpu.sync_copy(acc_sh.at[pl.ds(s * _SL, _SL)],
                        out_hbm.at[c, pl.ds(s * _SL, _SL)])

    return segsum


def _make_degree():
    @functools.partial(
        pl.kernel,
        out_type=jax.ShapeDtypeStruct((_NC, NPAD, F1), jnp.float32),
        mesh=_MESH,
        compiler_params=_SC_PARAMS,
        scratch_types=[
            pltpu.VMEM((_RPT, EW), jnp.int32),
            pltpu.VMEM((EW, F1), jnp.float32),
            pltpu.VMEM_SHARED((NPAD, F1), jnp.float32),
            pltpu.SemaphoreType.DMA,
        ],
    )
    def degree(ones_hbm, dst_hbm, zeros_hbm, out_hbm,
               dst_v, ones_v, acc_sh, ssem):
        c = lax.axis_index("c")
        s = lax.axis_index("s")
        wid = s * _NC + c
        pltpu.sync_copy(dst_hbm.at[pl.ds(wid * _RPT, _RPT)], dst_v)
        pltpu.sync_copy(ones_hbm.at[pl.ds(0, EW)], ones_v)
        pltpu.sync_copy(zeros_hbm.at[pl.ds(s * _SL, _SL)],
                        acc_sh.at[pl.ds(s * _SL, _SL)])
        plsc.subcore_barrier()

        @pl.loop(0, _RPT, step=_NB)
        def _(j0):
            ss = [pltpu.async_copy(ones_v, acc_sh.at[dst_v.at[j0 + b]],
                                   ssem, add=True)
                  for b in range(_NB)]
            for h in ss:
                h.wait()

        plsc.subcore_barrier()
        pltpu.sync_copy(acc_sh.at[pl.ds(s * _SL, _SL)],
                        out_hbm.at[c, pl.ds(s * _SL, _SL)])

    return degree


_segsum = _make_segsum()
_degree = _make_degree()


_FR = NPAD * F1 // 128
_FBR = 256
_FGRID = (_FR // _FBR,)


def _mm1_body(x_ref, w_ref, o_ref):
    o_ref[...] = jnp.dot(x_ref[...], w_ref[...],
                         preferred_element_type=jnp.float32)


def _scale_body(p_ref, h1_ref, dis_ref, h1s_ref):
    deg = p_ref[0] + p_ref[1] + 1.0
    dis = lax.rsqrt(deg)
    dis_ref[...] = dis
    h1s_ref[...] = h1_ref[...] * dis


def _layer1_body(p_ref, h1s_ref, dis_ref, b1_ref, r_ref):
    dis = dis_ref[...]
    o1 = dis * (p_ref[0] + p_ref[1] + h1s_ref[...]) + b1_ref[...]
    r_ref[...] = dis * jnp.maximum(o1, 0.0)


def _layer2_body(p_ref, r_ref, dis_ref, bd2_ref, b2_ref, kp_ref, kd_ref, o_ref):
    t = dis_ref[...] * (p_ref[0] + p_ref[1] + r_ref[...])
    o2 = jnp.dot(t, bd2_ref[...], preferred_element_type=jnp.float32) + b2_ref[...]
    ps = jnp.dot(o2, kp_ref[...], preferred_element_type=jnp.float32)
    pd = jnp.dot(o2, kd_ref[...], preferred_element_type=jnp.float32)
    m = 0.5 * (ps + jnp.abs(pd))
    es = jnp.dot(jnp.exp(o2 - m), kp_ref[...], preferred_element_type=jnp.float32)
    o_ref[...] = o2 - (m + jnp.log(es))


def _flat_spec(width=128):
    return pl.BlockSpec((_FBR, width), lambda i: (i, 0))


def _pairflat_spec():
    return pl.BlockSpec((_NC, _FBR, 128), lambda i: (0, i, 0))


def _const_spec(shape):
    return pl.BlockSpec(shape, lambda i: tuple(0 for _ in shape))


_FLAT_SDS = jax.ShapeDtypeStruct((_FR, 128), jnp.float32)


def kernel(x, edge_index, W1, b1, W2, b2):
    ei = edge_index.astype(jnp.int32)
    src = jnp.pad(ei[0], (0, EPAD - E), constant_values=DUMMY).reshape(ROWS, EW)
    dst = jnp.pad(ei[1], (0, EPAD - E), constant_values=DUMMY).reshape(ROWS, EW)
    x_pad = jnp.pad(x, ((0, NPAD - N), (0, 0)))
    zeros = jnp.zeros((NPAD, F1), jnp.float32)
    ones = jnp.ones((NPAD, F1), jnp.float32)

    eye8 = jnp.eye(8, dtype=jnp.float32)
    bd1 = jnp.kron(eye8, W1)
    bd2 = jnp.kron(eye8, W2.astype(jnp.float32))
    kp = jnp.kron(eye8, jnp.ones((F2, F2), jnp.float32))
    kd = jnp.kron(eye8, jnp.array([[1., -1.], [-1., 1.]], jnp.float32))
    b1t = jnp.tile(b1, 8).reshape(1, 128)
    b2t = jnp.tile(b2, 8).reshape(1, F1)

    degp = _degree(ones, dst, zeros)
    degp_f = degp.reshape(_NC, _FR, 128)

    h1_f = pl.pallas_call(
        _mm1_body,
        grid=_FGRID,
        in_specs=[pl.BlockSpec((_FBR, 8 * F0), lambda i: (i, 0)),
                  _const_spec((8 * F0, 128))],
        out_specs=_flat_spec(),
        out_shape=_FLAT_SDS,
    )(x_pad.reshape(_FR, 8 * F0), bd1)

    dis_f, h1s_f = pl.pallas_call(
        _scale_body,
        grid=_FGRID,
        in_specs=[_pairflat_spec(), _flat_spec()],
        out_specs=[_flat_spec(), _flat_spec()],
        out_shape=[_FLAT_SDS, _FLAT_SDS],
    )(degp_f, h1_f)

    a1p = _segsum(h1s_f.reshape(NPAD, F1), src, dst, zeros)

    r_f = pl.pallas_call(
        _layer1_body,
        grid=_FGRID,
        in_specs=[_pairflat_spec(), _flat_spec(), _flat_spec(),
                  _const_spec((1, 128))],
        out_specs=_flat_spec(),
        out_shape=_FLAT_SDS,
    )(a1p.reshape(_NC, _FR, 128), h1s_f, dis_f, b1t)

    a2p = _segsum(r_f.reshape(NPAD, F1), src, dst, zeros)

    out_f = pl.pallas_call(
        _layer2_body,
        grid=_FGRID,
        in_specs=[_pairflat_spec(), _flat_spec(), _flat_spec(),
                  _const_spec((128, F1)), _const_spec((1, F1)),
                  _const_spec((F1, F1)), _const_spec((F1, F1))],
        out_specs=pl.BlockSpec((_FBR, F1), lambda i: (i, 0)),
        out_shape=jax.ShapeDtypeStruct((_FR, F1), jnp.float32),
    )(a2p.reshape(_NC, _FR, 128), r_f, dis_f, bd2, b2t, kp, kd)

    return out_f.reshape(NPAD, F2)[:N]


# speedup vs baseline: 78.6762x; 1.0900x over previous
import functools

import jax
import jax.numpy as jnp
from jax import lax
from jax.experimental import pallas as pl
from jax.experimental.pallas import tpu as pltpu
from jax.experimental.pallas import tpu_sc as plsc

N = 10000
F0 = 128
F1 = 16
F2 = 2
E = 320000

NPAD = 10240
EW = 125
ROWS = E // EW

_NC = 2
_NS = 16
_RPT = ROWS // (_NC * _NS)
_SL = NPAD // _NS

_BR = 1024
_NB = 8


_MESH = plsc.VectorSubcoreMesh(core_axis_name="c", subcore_axis_name="s")
_SC_PARAMS = pltpu.CompilerParams(use_tc_tiling_on_sc=False)


def _make_segsum():
    @functools.partial(
        pl.kernel,
        out_type=jax.ShapeDtypeStruct((_NC, NPAD, F1), jnp.float32),
        mesh=_MESH,
        compiler_params=_SC_PARAMS,
        scratch_types=[
            pltpu.VMEM((_RPT, EW), jnp.int32),
            pltpu.VMEM((_RPT, EW), jnp.int32),
            pltpu.VMEM((_NB, EW, F1), jnp.float32),
            pltpu.VMEM_SHARED((NPAD, F1), jnp.float32),
            pltpu.VMEM_SHARED((NPAD, F1), jnp.float32),
            pltpu.SemaphoreType.DMA,
            pltpu.SemaphoreType.DMA,
        ],
    )
    def segsum(vals_hbm, src_hbm, dst_hbm, zeros_hbm, out_hbm,
               src_v, dst_v, rows_v, vals_sh, acc_sh, gsem, ssem):
        c = lax.axis_index("c")
        s = lax.axis_index("s")
        wid = s * _NC + c
        pltpu.sync_copy(src_hbm.at[pl.ds(wid * _RPT, _RPT)], src_v)
        pltpu.sync_copy(dst_hbm.at[pl.ds(wid * _RPT, _RPT)], dst_v)
        pltpu.sync_copy(vals_hbm.at[pl.ds(s * _SL, _SL)],
                        vals_sh.at[pl.ds(s * _SL, _SL)])
        pltpu.sync_copy(zeros_hbm.at[pl.ds(s * _SL, _SL)],
                        acc_sh.at[pl.ds(s * _SL, _SL)])
        plsc.subcore_barrier()

        @pl.loop(0, _RPT, step=_NB)
        def _(j0):
            gs = [pltpu.async_copy(vals_sh.at[src_v.at[j0 + b]],
                                   rows_v.at[b], gsem)
                  for b in range(_NB)]
            ss = []
            for b in range(_NB):
                gs[b].wait()
                ss.append(pltpu.async_copy(rows_v.at[b],
                                           acc_sh.at[dst_v.at[j0 + b]],
                                           ssem, add=True))
            for h in ss:
                h.wait()

        plsc.subcore_barrier()
        pltpu.sync_copy(acc_sh.at[pl.ds(s * _SL, _SL)],
                        out_hbm.at[c, pl.ds(s * _SL, _SL)])

    return segsum


def _make_degree():
    @functools.partial(
        pl.kernel,
        out_type=jax.ShapeDtypeStruct((_NC, NPAD, F1), jnp.float32),
        mesh=_MESH,
        compiler_params=_SC_PARAMS,
        scratch_types=[
            pltpu.VMEM((_RPT, EW), jnp.int32),
            pltpu.VMEM((EW, F1), jnp.float32),
            pltpu.VMEM_SHARED((NPAD, F1), jnp.float32),
            pltpu.SemaphoreType.DMA,
        ],
    )
    def degree(ones_hbm, dst_hbm, zeros_hbm, out_hbm,
               dst_v, ones_v, acc_sh, ssem):
        c = lax.axis_index("c")
        s = lax.axis_index("s")
        wid = s * _NC + c
        pltpu.sync_copy(dst_hbm.at[pl.ds(wid * _RPT, _RPT)], dst_v)
        pltpu.sync_copy(ones_hbm.at[pl.ds(0, EW)], ones_v)
        pltpu.sync_copy(zeros_hbm.at[pl.ds(s * _SL, _SL)],
                        acc_sh.at[pl.ds(s * _SL, _SL)])
        plsc.subcore_barrier()

        @pl.loop(0, _RPT, step=_NB)
        def _(j0):
            ss = [pltpu.async_copy(ones_v, acc_sh.at[dst_v.at[j0 + b]],
                                   ssem, add=True)
                  for b in range(_NB)]
            for h in ss:
                h.wait()

        plsc.subcore_barrier()
        pltpu.sync_copy(acc_sh.at[pl.ds(s * _SL, _SL)],
                        out_hbm.at[c, pl.ds(s * _SL, _SL)])

    return degree


_segsum = _make_segsum()
_degree = _make_degree()


_FR = NPAD * F1 // 128
_FBR = 256
_FGRID = (_FR // _FBR,)


def _mm1_body(x_ref, w_ref, o_ref):
    o_ref[...] = jnp.dot(x_ref[...], w_ref[...],
                         preferred_element_type=jnp.float32)


def _scale_body(p_ref, h1_ref, dis_ref, h1s_ref):
    deg = p_ref[0] + p_ref[1] + 1.0
    dis = lax.rsqrt(deg)
    dis_ref[...] = dis
    h1s_ref[...] = h1_ref[...] * dis


def _layer1_body(p_ref, h1s_ref, dis_ref, b1_ref, r_ref):
    dis = dis_ref[...]
    o1 = dis * (p_ref[0] + p_ref[1] + h1s_ref[...]) + b1_ref[...]
    r_ref[...] = dis * jnp.maximum(o1, 0.0)


def _layer2_body(p_ref, r_ref, dis_ref, bd2_ref, b2_ref, kp_ref, kd_ref, o_ref):
    t = dis_ref[...] * (p_ref[0] + p_ref[1] + r_ref[...])
    o2 = jnp.dot(t, bd2_ref[...], preferred_element_type=jnp.float32) + b2_ref[...]
    ps = jnp.dot(o2, kp_ref[...], preferred_element_type=jnp.float32)
    pd = jnp.dot(o2, kd_ref[...], preferred_element_type=jnp.float32)
    m = 0.5 * (ps + jnp.abs(pd))
    es = jnp.dot(jnp.exp(o2 - m), kp_ref[...], preferred_element_type=jnp.float32)
    o_ref[...] = o2 - (m + jnp.log(es))


def _flat_spec(width=128):
    return pl.BlockSpec((_FBR, width), lambda i: (i, 0))


def _pairflat_spec():
    return pl.BlockSpec((_NC, _FBR, 128), lambda i: (0, i, 0))


def _const_spec(shape):
    return pl.BlockSpec(shape, lambda i: tuple(0 for _ in shape))


_FLAT_SDS = jax.ShapeDtypeStruct((_FR, 128), jnp.float32)


def kernel(x, edge_index, W1, b1, W2, b2):
    ei = edge_index.astype(jnp.int32)
    src = ei[0].reshape(ROWS, EW)
    dst = ei[1].reshape(ROWS, EW)
    x_pad = jnp.pad(x, ((0, NPAD - N), (0, 0)))
    zeros = jnp.zeros((NPAD, F1), jnp.float32)
    ones = jnp.ones((NPAD, F1), jnp.float32)

    eye8 = jnp.eye(8, dtype=jnp.float32)
    bd1 = jnp.kron(eye8, W1)
    bd2 = jnp.kron(eye8, W2.astype(jnp.float32))
    kp = jnp.kron(eye8, jnp.ones((F2, F2), jnp.float32))
    kd = jnp.kron(eye8, jnp.array([[1., -1.], [-1., 1.]], jnp.float32))
    b1t = jnp.tile(b1, 8).reshape(1, 128)
    b2t = jnp.tile(b2, 8).reshape(1, F1)

    degp = _degree(ones, dst, zeros)
    degp_f = degp.reshape(_NC, _FR, 128)

    h1_f = pl.pallas_call(
        _mm1_body,
        grid=_FGRID,
        in_specs=[pl.BlockSpec((_FBR, 8 * F0), lambda i: (i, 0)),
                  _const_spec((8 * F0, 128))],
        out_specs=_flat_spec(),
        out_shape=_FLAT_SDS,
    )(x_pad.reshape(_FR, 8 * F0), bd1)

    dis_f, h1s_f = pl.pallas_call(
        _scale_body,
        grid=_FGRID,
        in_specs=[_pairflat_spec(), _flat_spec()],
        out_specs=[_flat_spec(), _flat_spec()],
        out_shape=[_FLAT_SDS, _FLAT_SDS],
    )(degp_f, h1_f)

    a1p = _segsum(h1s_f.reshape(NPAD, F1), src, dst, zeros)

    r_f = pl.pallas_call(
        _layer1_body,
        grid=_FGRID,
        in_specs=[_pairflat_spec(), _flat_spec(), _flat_spec(),
                  _const_spec((1, 128))],
        out_specs=_flat_spec(),
        out_shape=_FLAT_SDS,
    )(a1p.reshape(_NC, _FR, 128), h1s_f, dis_f, b1t)

    a2p = _segsum(r_f.reshape(NPAD, F1), src, dst, zeros)

    out_f = pl.pallas_call(
        _layer2_body,
        grid=_FGRID,
        in_specs=[_pairflat_spec(), _flat_spec(), _flat_spec(),
                  _const_spec((128, F1)), _const_spec((1, F1)),
                  _const_spec((F1, F1)), _const_spec((F1, F1))],
        out_specs=pl.BlockSpec((_FBR, F1), lambda i: (i, 0)),
        out_shape=jax.ShapeDtypeStruct((_FR, F1), jnp.float32),
    )(a2p.reshape(_NC, _FR, 128), r_f, dis_f, bd2, b2t, kp, kd)

    return out_f.reshape(NPAD, F2)[:N]
